# baseline (speedup 1.0000x reference)
import functools

import jax
import jax.numpy as jnp
import numpy as np
from jax import lax
from jax.experimental import pallas as pl
from jax.experimental.pallas import tpu as pltpu
from jax.experimental.pallas import tpu_sc as plsc

_FIELD_DIMS = [100000] * 26
_F = len(_FIELD_DIMS)
_D = 16
_B = 16384
_L = 16
_NC, _NS = 2, 16
_NW = _NC * _NS
_BPW = _B // _NW
_CH = 8
_RPC = _BPW // _CH
_IPC = _RPC * _F
_IW = 128
_NG = _IPC // _IW
_NROW = _CH * _NG


def _body(x_hbm, off_hbm, table_hbm, out_hbm,
          idx_v, off_v, buf0, buf1, out_v, sem0, sem1):
    wid = lax.axis_index("s") * _NC + lax.axis_index("c")

    pltpu.sync_copy(x_hbm.at[wid], idx_v)
    pltpu.sync_copy(off_hbm, off_v)

    bufs = (buf0, buf1)
    sems = (sem0, sem1)
    inv_f = jnp.float32(1.0 / _F)

    def adjust_chunk(c):
        def row_body(g, carry):
            for k in range(_IW // _L):
                sl = pl.ds(k * _L, _L)
                idx_v[c * _NG + g, sl] = idx_v[c * _NG + g, sl] + off_v[g, sl]
            return carry
        lax.fori_loop(0, _NG, row_body, 0, unroll=True)

    def fire_chunk(c):
        buf = bufs[c % 2]
        sem = sems[c % 2]
        handles = []
        for g in range(_NG):
            cp = pltpu.async_copy(
                table_hbm.at[idx_v.at[c * _NG + g]],
                buf.at[pl.ds(g * _IW, _IW), :],
                sem)
            handles.append(cp)
        return handles

    def reduce_chunk(c):
        buf = bufs[c % 2]

        def row_body(r, carry):
            base = r * _F
            acc = buf[base, :]
            for j in range(1, _F):
                acc = acc + buf[base + j, :]
            out_v[c * _RPC + r, :] = acc * inv_f
            return carry
        lax.fori_loop(0, _RPC, row_body, 0)

    adjust_chunk(0)
    pending = fire_chunk(0)
    for c in range(_CH):
        if c + 1 < _CH:
            adjust_chunk(c + 1)
            nxt = fire_chunk(c + 1)
        else:
            nxt = []
        for cp in pending:
            cp.wait()
        reduce_chunk(c)
        pending = nxt

    pltpu.sync_copy(out_v, out_hbm.at[pl.ds(wid * _BPW, _BPW)])


@jax.jit
def _embedding_bag(x_blocked, off, table):
    mesh = plsc.VectorSubcoreMesh(core_axis_name="c", subcore_axis_name="s")
    k = pl.kernel(
        _body,
        out_type=jax.ShapeDtypeStruct((_B, _D), jnp.float32),
        mesh=mesh,
        scratch_types=[
            pltpu.VMEM((_NROW, _IW), jnp.int32),
            pltpu.VMEM((_NG, _IW), jnp.int32),
            pltpu.VMEM((_IPC, _D), jnp.float32),
            pltpu.VMEM((_IPC, _D), jnp.float32),
            pltpu.VMEM((_BPW, _D), jnp.float32),
            pltpu.SemaphoreType.DMA,
            pltpu.SemaphoreType.DMA,
        ],
        compiler_params=pltpu.CompilerParams(use_tc_tiling_on_sc=False),
    )
    return k(x_blocked, off, table)


def kernel(x, table):
    x = x.astype(jnp.int32).reshape(_NW, _NROW, _IW)
    offsets = np.concatenate([[0], np.cumsum(_FIELD_DIMS)[:-1]]).astype(np.int32)
    off = jnp.asarray(np.tile(offsets, _IPC // _F).reshape(_NG, _IW))
    return _embedding_bag(x, off, table)
